# baseline (speedup 1.0000x reference)
import jax
import jax.numpy as jnp
from jax import lax
from jax.experimental import pallas as pl
from jax.experimental.pallas import tpu as pltpu
from jax.experimental.pallas import tpu_sc as plsc

_NUM_CLASSES = 20
_EPS = 1e-12
_NW = 32
_D = 128
_STRIDE = 160
_BINS = _NUM_CLASSES * _STRIDE
_R = 256


def _rsqrt_vec(ssv):
    f = lambda v: jnp.full((16,), v, jnp.float32)
    ssg = jnp.maximum(ssv, f(1e-34))
    i = lax.bitcast_convert_type(ssg, jnp.int32)
    magic = jnp.full((16,), 0x5F3759DF, jnp.int32)
    one = jnp.full((16,), 1, jnp.int32)
    y = lax.bitcast_convert_type(
        magic - lax.shift_right_arithmetic(i, one), jnp.float32)
    c15, c05 = f(1.5), f(0.5)
    for _ in range(3):
        y = y * (c15 - c05 * ssg * y * y)
    return jnp.minimum(y, f(1e12))


def _sc_body(p1_ref, p2_ref, l1_ref, l2_ref, out_ref, pbuf, lbuf, bins):
    cid = lax.axis_index("c")
    sid = lax.axis_index("s")
    wid = sid * 2 + cid
    rw = p1_ref.shape[0] // _D // _NW
    nchunks = rw // _R
    zeros16 = jnp.zeros((16,), jnp.float32)
    iota16 = lax.broadcasted_iota(jnp.int32, (16,), 0)
    ones_slot = jnp.where(iota16 == jnp.zeros((16,), jnp.int32),
                          jnp.ones((16,), jnp.float32), zeros16)

    for b, (p_ref, l_ref) in enumerate(((p1_ref, l1_ref), (p2_ref, l2_ref))):
        row0 = wid * rw

        def _zero(i, _):
            bins[pl.ds(i * 16, 16)] = zeros16
            return 0
        lax.fori_loop(0, _BINS // 16, _zero, 0)

        def _chunk(c, _c):
            base = row0 + c * _R
            pltpu.sync_copy(p_ref.at[pl.ds(base * _D, _R * _D)], pbuf)
            pltpu.sync_copy(l_ref.at[pl.ds(base, _R)], lbuf)

            def _group(g, _g):
                lv = lbuf[pl.ds(g * 16, 16)]
                for k in range(16):
                    off = (g * 16 + k) * _D
                    lk = lv[k]
                    vs = [pbuf[pl.ds(off + j * 16, 16)] for j in range(8)]
                    acc = vs[0] * vs[0]
                    for j in range(1, 8):
                        acc = acc + vs[j] * vs[j]
                    ssv = lax.broadcast_in_dim(jnp.sum(acc), (16,), ())
                    rinv = _rsqrt_vec(ssv)
                    bbase = lk * _STRIDE
                    for j in range(8):
                        plsc.addupdate(bins.at[pl.ds(bbase + j * 16, 16)],
                                       vs[j] * rinv)
                    plsc.addupdate(bins.at[pl.ds(bbase + _D, 16)], ones_slot)
                return 0
            lax.fori_loop(0, _R // 16, _group, 0)
            return 0
        lax.fori_loop(0, nchunks, _chunk, 0)
        pltpu.sync_copy(bins,
                        out_ref.at[pl.ds((b * _NW + wid) * _BINS, _BINS)])


def _sc_segment_sums(p1f, p2f, labels1, labels2):
    kfn = pl.kernel(
        _sc_body,
        out_type=jax.ShapeDtypeStruct((2 * _NW * _BINS,), jnp.float32),
        mesh=plsc.VectorSubcoreMesh(core_axis_name="c", subcore_axis_name="s"),
        compiler_params=pltpu.CompilerParams(needs_layout_passes=False),
        scratch_types=[
            pltpu.VMEM((_R * _D,), jnp.float32),
            pltpu.VMEM((_R,), jnp.int32),
            pltpu.VMEM((_BINS,), jnp.float32),
        ],
    )
    return kfn(p1f, p2f, labels1, labels2)


def _epi_body(a_ref, sp_ref, loss_ref, sl1_ref, sl2_ref, si1_ref, si2_ref):
    a = a_ref[...]
    an = a / jnp.maximum(jnp.sqrt(jnp.sum(a * a, axis=1, keepdims=True)),
                         _EPS)

    def branch(b):
        s = jnp.zeros((_NUM_CLASSES, _D), jnp.float32)
        cnt = jnp.zeros((_NUM_CLASSES, 1), jnp.float32)
        for w in range(_NW):
            r0 = (b * _NW + w) * _NUM_CLASSES
            blk = sp_ref[r0:r0 + _NUM_CLASSES, :]
            s = s + blk[:, 0:_D]
            cnt = cnt + blk[:, _D:_D + 1]
        dots = jnp.sum(s * an, axis=1, keepdims=True)
        sl = (cnt - dots) * 0.25
        mean = sl / jnp.maximum(cnt, 1.0)
        present = (cnt > 0).astype(jnp.float32)
        bal = (jnp.sum(mean * present, axis=0, keepdims=True) /
               jnp.maximum(jnp.sum(present, axis=0, keepdims=True), 1.0))
        return sl, cnt, bal

    sl1, c1, b1 = branch(0)
    sl2, c2, b2 = branch(1)
    loss_ref[...] = b1 + b2
    sl1_ref[...] = sl1
    si1_ref[...] = c1
    sl2_ref[...] = sl2
    si2_ref[...] = c2


def kernel(p1, p2, z1, z2, anchor_features, corrs1, corrs2, labels1, labels2):
    f32 = jnp.float32
    sp = _sc_segment_sums(p1.reshape(-1), p2.reshape(-1), labels1, labels2)
    sp2 = sp.reshape(2 * _NW * _NUM_CLASSES, _STRIDE)
    outs = pl.pallas_call(
        _epi_body,
        out_shape=[
            jax.ShapeDtypeStruct((1, 1), f32),
            jax.ShapeDtypeStruct((_NUM_CLASSES, 1), f32),
            jax.ShapeDtypeStruct((_NUM_CLASSES, 1), f32),
            jax.ShapeDtypeStruct((_NUM_CLASSES, 1), f32),
            jax.ShapeDtypeStruct((_NUM_CLASSES, 1), f32),
        ],
    )(anchor_features, sp2)
    loss, sl1, sl2, si1, si2 = outs
    return (loss[0, 0], sl1[:, 0], sl2[:, 0], si1[:, 0], si2[:, 0])
